# Initial kernel scaffold; baseline (speedup 1.0000x reference)
import jax
import jax.numpy as jnp
from jax.experimental import pallas as pl


def kernel(user_idx, item_idx, emb, W1, W2, L_row, L_col, L_val):
    raise NotImplementedError("write your pallas kernel here")


# speedup vs baseline: 3.3666x; 3.3666x over previous
import functools

import jax
import jax.numpy as jnp
from jax import lax
from jax.experimental import pallas as pl
from jax.experimental.pallas import tpu as pltpu
from jax.experimental.pallas import tpu_sc as plsc

D = 64
NEG = 0.2
NC, NS, L = 2, 16, 16

K = 256
SUB = 128
NSUB = K // SUB

TRASH = 512


def _spmm_kernel(n_node, nnz_pad):
    half = n_node // NC
    ept = nnz_pad // NS
    nchunk = ept // K
    acc_rows = half + TRASH
    acc_rows += (-acc_rows) % NS
    stripe = acc_rows // NS
    rpt = ((half + NS - 1) // NS + 7) // 8 * 8
    rpt_last = half - (NS - 1) * rpt
    assert rpt_last > 0 and rpt_last % 8 == 0

    def _chunks(total):
        out, off = [], 0
        while off < total:
            out.append((off, min(K, total - off)))
            off += out[-1][1]
        return out
    zchunks = _chunks(stripe)

    mesh = plsc.VectorSubcoreMesh(core_axis_name="c", subcore_axis_name="s", num_cores=NC, num_subcores=NS)

    @functools.partial(
        pl.kernel,
        out_type=jax.ShapeDtypeStruct((n_node, D), jnp.float32),
        mesh=mesh,
        compiler_params=pltpu.CompilerParams(needs_layout_passes=False, use_tc_tiling_on_sc=False),
        scratch_types=[
            pltpu.VMEM((K, D), jnp.float32),
            pltpu.VMEM((NSUB, SUB), jnp.int32),
            pltpu.VMEM((NSUB, SUB), jnp.int32),
            pltpu.VMEM((NSUB, SUB), jnp.int32),
            pltpu.VMEM((K,), jnp.float32),
            pltpu.VMEM_SHARED((acc_rows, D), jnp.float32),
            pltpu.SemaphoreType.DMA,
        ],
    )
    def spmm(e_hbm, col_hbm, row_hbm, val_hbm, out_hbm,
             rows_v, col_v, row_v, loc_v, val_v, acc, sem):
        cid = lax.axis_index("c")
        sid = lax.axis_index("s")
        base_row = cid * half

        def zrow(r, _):
            for j in range(D // L):
                rows_v[r, pl.ds(j * L, L)] = jnp.zeros((L,), jnp.float32)
            return 0
        lax.fori_loop(0, K, zrow, 0)
        for zoff, zn in zchunks:
            pltpu.sync_copy(rows_v.at[pl.ds(0, zn)],
                            acc.at[pl.ds(sid * stripe + zoff, zn)])
        plsc.subcore_barrier()

        def chunk(g, _):
            b2 = sid * (ept // SUB) + g * NSUB
            pltpu.sync_copy(col_hbm.at[pl.ds(b2, NSUB)], col_v)
            pltpu.sync_copy(row_hbm.at[pl.ds(b2, NSUB)], row_v)
            pltpu.sync_copy(val_hbm.at[pl.ds(sid * ept + g * K, K)], val_v)

            cps = [pltpu.async_copy(e_hbm.at[col_v.at[s]],
                                    rows_v.at[pl.ds(s * SUB, SUB)], sem)
                   for s in range(NSUB)]
            for cp in cps:
                cp.wait()

            def rloc(i, _):
                s = i // (SUB // L)
                o = (i % (SUB // L)) * L
                r = row_v[s, pl.ds(o, L)]
                ok = (r >= base_row) & (r < base_row + half)
                spread = half + ((lax.iota(jnp.int32, 16) + i * L) & (TRASH - 1))
                loc_v[s, pl.ds(o, L)] = jnp.where(ok, r - base_row, spread)
                return 0
            lax.fori_loop(0, K // L, rloc, 0)

            def scale(e, _):
                v = plsc.load_gather(val_v, [jnp.full((L,), e, jnp.int32)])
                for j in range(D // L):
                    rows_v[e, pl.ds(j * L, L)] = rows_v[e, pl.ds(j * L, L)] * v
                return 0
            lax.fori_loop(0, K, scale, 0, unroll=2)

            for s in range(NSUB):
                pltpu.sync_copy(rows_v.at[pl.ds(s * SUB, SUB)],
                                acc.at[loc_v.at[s]], add=True)
            return 0
        lax.fori_loop(0, nchunk, chunk, 0)
        plsc.subcore_barrier()

        def copy_out(total):
            for coff, cn in _chunks(total):
                start = sid * rpt + coff
                pltpu.sync_copy(acc.at[pl.ds(start, cn)],
                                rows_v.at[pl.ds(0, cn)])
                pltpu.sync_copy(rows_v.at[pl.ds(0, cn)],
                                out_hbm.at[pl.ds(base_row + start, cn)])
        @pl.when(sid < NS - 1)
        def _():
            copy_out(rpt)
        @pl.when(sid == NS - 1)
        def _():
            copy_out(rpt_last)

    return spmm


def _dense_kernel(n_node):
    blk = 2000
    assert n_node % blk == 0

    def body(side_ref, e_ref, w1_ref, w2_ref, out_ref):
        s = side_ref[...]
        e = e_ref[...]
        y = (jnp.dot(s + e, w1_ref[...], preferred_element_type=jnp.float32)
             + jnp.dot(e * s, w2_ref[...], preferred_element_type=jnp.float32))
        out_ref[...] = jnp.where(y >= 0, y, NEG * y)

    return pl.pallas_call(
        body,
        grid=(n_node // blk,),
        in_specs=[
            pl.BlockSpec((blk, D), lambda i: (i, 0)),
            pl.BlockSpec((blk, D), lambda i: (i, 0)),
            pl.BlockSpec((D, D), lambda i: (0, 0)),
            pl.BlockSpec((D, D), lambda i: (0, 0)),
        ],
        out_specs=pl.BlockSpec((blk, D), lambda i: (i, 0)),
        out_shape=jax.ShapeDtypeStruct((n_node, D), jnp.float32),
    )


def _gather_kernel(n_node, n_idx, n_tab):
    nw = NC * NS
    per_w = n_idx // nw
    gsub = per_w // SUB
    assert per_w % SUB == 0

    mesh = plsc.VectorSubcoreMesh(core_axis_name="c", subcore_axis_name="s", num_cores=NC, num_subcores=NS)

    @functools.partial(
        pl.kernel,
        out_type=jax.ShapeDtypeStruct((n_tab, n_idx, D), jnp.float32),
        mesh=mesh,
        compiler_params=pltpu.CompilerParams(needs_layout_passes=False,
                                             use_tc_tiling_on_sc=False),
        scratch_types=[
            pltpu.VMEM((gsub, SUB), jnp.int32),
            pltpu.VMEM((per_w, D), jnp.float32),
            pltpu.SemaphoreType.DMA,
        ],
    )
    def gather(t0, t1, t2, t3, idx_hbm, out_hbm, idx_v, rows_v, sem):
        wid = lax.axis_index("s") * NC + lax.axis_index("c")
        base = wid * per_w
        pltpu.sync_copy(idx_hbm.at[pl.ds(wid * gsub, gsub)], idx_v)
        for t, tab in enumerate((t0, t1, t2, t3)):
            cps = [pltpu.async_copy(tab.at[idx_v.at[s]],
                                    rows_v.at[pl.ds(s * SUB, SUB)], sem)
                   for s in range(gsub)]
            for cp in cps:
                cp.wait()
            pltpu.sync_copy(rows_v, out_hbm.at[t, pl.ds(base, per_w)])

    return gather


def kernel(user_idx, item_idx, emb, W1, W2, L_row, L_col, L_val):
    n_node, d = emb.shape
    n_layer = W1.shape[0]
    nnz = L_row.shape[0]
    assert d == D

    nnz_pad = -(-nnz // (NS * K)) * (NS * K)
    pad = nnz_pad - nnz
    if pad:
        spread = (jnp.arange(pad, dtype=jnp.int32) * 67) % n_node
        L_row = jnp.concatenate([L_row, spread])
        L_col = jnp.concatenate([L_col, spread])
        L_val = jnp.concatenate([L_val, jnp.zeros((pad,), jnp.float32)])
    col2 = L_col.reshape(-1, SUB)
    row2 = L_row.reshape(-1, SUB)
    val2 = L_val

    spmm = _spmm_kernel(n_node, nnz_pad)
    dense = _dense_kernel(n_node)

    e_prev = emb
    e_list = [emb]
    for layer in range(n_layer):
        side = spmm(e_prev, col2, row2, val2)
        e_prev = dense(side, e_prev, W1[layer], W2[layer])
        e_list.append(e_prev)

    n_users = n_node // 2
    batch = user_idx.shape[0]
    all_idx = jnp.concatenate([user_idx, item_idx + n_users]).reshape(-1, SUB)
    g = _gather_kernel(n_node, 2 * batch, len(e_list))(*e_list, all_idx)
    e_user = jnp.concatenate([g[t, :batch] for t in range(len(e_list))], axis=1)
    e_item = jnp.concatenate([g[t, batch:] for t in range(len(e_list))], axis=1)
    return (e_user, e_item)


# speedup vs baseline: 7.5702x; 2.2486x over previous
import functools

import jax
import jax.numpy as jnp
from jax import lax
from jax.experimental import pallas as pl
from jax.experimental.pallas import tpu as pltpu
from jax.experimental.pallas import tpu_sc as plsc

D = 64
NEG = 0.2
NC, NS, L = 2, 16, 16

K = 128
SUB = 128
NSUB = K // SUB

TRASH = 512


NB = 3


def _spmm_kernel(n_node, nnz_pad):
    half = n_node // NC
    ept = nnz_pad // NS
    nchunk = ept // K
    nmacro = nchunk // NB
    assert nchunk % NB == 0 and ept % 8 == 0
    acc_rows = half + TRASH
    acc_rows += (-acc_rows) % NS
    stripe = acc_rows // NS
    rpt = ((half + NS - 1) // NS + 7) // 8 * 8
    rpt_last = half - (NS - 1) * rpt
    assert rpt_last > 0 and rpt_last % 8 == 0

    def _chunks(total):
        out, off = [], 0
        while off < total:
            out.append((off, min(K, total - off)))
            off += out[-1][1]
        return out

    mesh = plsc.VectorSubcoreMesh(core_axis_name="c", subcore_axis_name="s",
                                  num_cores=NC, num_subcores=NS)

    @functools.partial(
        pl.kernel,
        out_type=jax.ShapeDtypeStruct((n_node, D), jnp.float32),
        mesh=mesh,
        compiler_params=pltpu.CompilerParams(needs_layout_passes=False,
                                             use_tc_tiling_on_sc=False),
        scratch_types=[
            pltpu.VMEM((NB, K, D), jnp.float32),
            pltpu.VMEM((NB, 3, K), jnp.int32),
            pltpu.VMEM((NB, K), jnp.int32),
            pltpu.VMEM_SHARED((acc_rows, D), jnp.float32),
        ] + [pltpu.SemaphoreType.DMA] * (3 * NB),
    )
    def spmm(e_hbm, coo_hbm, out_hbm, rows_v, coo_v, loc_v, acc, *sems):
        semc, semg, sems_ = sems[0:NB], sems[NB:2 * NB], sems[2 * NB:3 * NB]
        cid = lax.axis_index("c")
        sid = lax.axis_index("s")
        base_row = cid * half
        cbase = sid * nchunk

        def zrow(r, _):
            for j in range(D // L):
                rows_v[0, r, pl.ds(j * L, L)] = jnp.zeros((L,), jnp.float32)
            return 0
        lax.fori_loop(0, K, zrow, 0)
        for zoff, zn in _chunks(stripe):
            pltpu.sync_copy(rows_v.at[0, pl.ds(0, zn)],
                            acc.at[pl.ds(sid * stripe + zoff, zn)])
        plsc.subcore_barrier()

        def process(b, g):
            def rloc(i, _):
                r = coo_v[b, 1, pl.ds(i * L, L)]
                ok = (r >= base_row) & (r < base_row + half)
                spread = half + ((lax.iota(jnp.int32, L) + i * L + g * 13)
                                 & (TRASH - 1))
                loc_v[b, pl.ds(i * L, L)] = jnp.where(ok, r - base_row, spread)
                return 0
            lax.fori_loop(0, K // L, rloc, 0)

            def scale(e, _):
                v = plsc.bitcast(
                    plsc.load_gather(coo_v, [jnp.full((L,), b, jnp.int32),
                                             jnp.full((L,), 2, jnp.int32),
                                             jnp.full((L,), e, jnp.int32)]),
                    jnp.float32)
                for j in range(D // L):
                    rows_v[b, e, pl.ds(j * L, L)] = (
                        rows_v[b, e, pl.ds(j * L, L)] * v)
                return 0
            lax.fori_loop(0, K, scale, 0, unroll=2)
            pltpu.async_copy(rows_v.at[b], acc.at[loc_v.at[b]], sems_[b],
                             add=True)

        pltpu.async_copy(coo_hbm.at[cbase], coo_v.at[0], semc[0])
        pltpu.async_copy(coo_hbm.at[cbase + 1], coo_v.at[1], semc[1])

        def macro(g2, _):
            for b in range(NB):
                g = g2 * NB + b
                pb = (b + NB - 1) % NB
                pltpu.make_async_copy(coo_hbm.at[cbase],
                                      coo_v.at[b], semc[b]).wait()
                @pl.when(g2 >= 1)
                def _():
                    pltpu.make_async_copy(rows_v.at[b], acc.at[loc_v.at[b]],
                                          sems_[b]).wait()
                pltpu.async_copy(e_hbm.at[coo_v.at[b, 0]], rows_v.at[b],
                                 semg[b])
                if b == 0:
                    @pl.when(g2 >= 1)
                    def _():
                        pltpu.make_async_copy(e_hbm.at[coo_v.at[pb, 0]],
                                              rows_v.at[pb], semg[pb]).wait()
                        process(pb, g - 1)
                else:
                    pltpu.make_async_copy(e_hbm.at[coo_v.at[pb, 0]],
                                          rows_v.at[pb], semg[pb]).wait()
                    process(pb, g - 1)
                nxt = lax.min(cbase + g + 2, cbase + nchunk - 1)
                if b == 0:
                    pltpu.async_copy(coo_hbm.at[nxt], coo_v.at[pb], semc[pb])
                else:
                    @pl.when(g2 < nmacro - 1)
                    def _():
                        pltpu.async_copy(coo_hbm.at[nxt], coo_v.at[pb],
                                         semc[pb])
            return 0
        lax.fori_loop(0, nmacro, macro, 0)

        lb = (nchunk - 1) % NB
        pltpu.make_async_copy(e_hbm.at[coo_v.at[lb, 0]], rows_v.at[lb],
                              semg[lb]).wait()
        process(lb, nchunk - 1)
        for b in range(NB):
            pltpu.make_async_copy(rows_v.at[b], acc.at[loc_v.at[b]],
                                  sems_[b]).wait()
        plsc.subcore_barrier()

        def copy_out(total):
            for coff, cn in _chunks(total):
                start = sid * rpt + coff
                pltpu.sync_copy(acc.at[pl.ds(start, cn)],
                                rows_v.at[0, pl.ds(0, cn)])
                pltpu.sync_copy(rows_v.at[0, pl.ds(0, cn)],
                                out_hbm.at[pl.ds(base_row + start, cn)])
        @pl.when(sid < NS - 1)
        def _():
            copy_out(rpt)
        @pl.when(sid == NS - 1)
        def _():
            copy_out(rpt_last)

    return spmm


def _dense_kernel(n_node):
    blk = 2000
    assert n_node % blk == 0

    def body(side_ref, e_ref, w1_ref, w2_ref, out_ref):
        s = side_ref[...]
        e = e_ref[...]
        y = (jnp.dot(s + e, w1_ref[...], preferred_element_type=jnp.float32)
             + jnp.dot(e * s, w2_ref[...], preferred_element_type=jnp.float32))
        out_ref[...] = jnp.where(y >= 0, y, NEG * y)

    return pl.pallas_call(
        body,
        grid=(n_node // blk,),
        in_specs=[
            pl.BlockSpec((blk, D), lambda i: (i, 0)),
            pl.BlockSpec((blk, D), lambda i: (i, 0)),
            pl.BlockSpec((D, D), lambda i: (0, 0)),
            pl.BlockSpec((D, D), lambda i: (0, 0)),
        ],
        out_specs=pl.BlockSpec((blk, D), lambda i: (i, 0)),
        out_shape=jax.ShapeDtypeStruct((n_node, D), jnp.float32),
    )


def _gather_kernel(n_node, n_idx, n_tab):
    nw = NC * NS
    per_w = n_idx // nw
    gsub = per_w // SUB
    assert per_w % SUB == 0

    mesh = plsc.VectorSubcoreMesh(core_axis_name="c", subcore_axis_name="s", num_cores=NC, num_subcores=NS)

    @functools.partial(
        pl.kernel,
        out_type=jax.ShapeDtypeStruct((n_tab, n_idx, D), jnp.float32),
        mesh=mesh,
        compiler_params=pltpu.CompilerParams(needs_layout_passes=False,
                                             use_tc_tiling_on_sc=False),
        scratch_types=[
            pltpu.VMEM((gsub, SUB), jnp.int32),
            pltpu.VMEM((per_w, D), jnp.float32),
            pltpu.SemaphoreType.DMA,
        ],
    )
    def gather(t0, t1, t2, t3, idx_hbm, out_hbm, idx_v, rows_v, sem):
        wid = lax.axis_index("s") * NC + lax.axis_index("c")
        base = wid * per_w
        pltpu.sync_copy(idx_hbm.at[pl.ds(wid * gsub, gsub)], idx_v)
        for t, tab in enumerate((t0, t1, t2, t3)):
            cps = [pltpu.async_copy(tab.at[idx_v.at[s]],
                                    rows_v.at[pl.ds(s * SUB, SUB)], sem)
                   for s in range(gsub)]
            for cp in cps:
                cp.wait()
            pltpu.sync_copy(rows_v, out_hbm.at[t, pl.ds(base, per_w)])

    return gather


def kernel(user_idx, item_idx, emb, W1, W2, L_row, L_col, L_val):
    n_node, d = emb.shape
    n_layer = W1.shape[0]
    nnz = L_row.shape[0]
    assert d == D

    nnz_pad = -(-nnz // (NS * K * NB)) * (NS * K * NB)
    pad = nnz_pad - nnz
    if pad:
        spread = (jnp.arange(pad, dtype=jnp.int32) * 67) % n_node
        L_row = jnp.concatenate([L_row, spread])
        L_col = jnp.concatenate([L_col, spread])
        L_val = jnp.concatenate([L_val, jnp.zeros((pad,), jnp.float32)])
    val_bits = lax.bitcast_convert_type(L_val, jnp.int32)
    coo = jnp.stack([L_col.reshape(-1, K), L_row.reshape(-1, K),
                     val_bits.reshape(-1, K)], axis=1)

    spmm = _spmm_kernel(n_node, nnz_pad)
    dense = _dense_kernel(n_node)

    e_prev = emb
    e_list = [emb]
    for layer in range(n_layer):
        side = spmm(e_prev, coo)
        e_prev = dense(side, e_prev, W1[layer], W2[layer])
        e_list.append(e_prev)

    n_users = n_node // 2
    batch = user_idx.shape[0]
    all_idx = jnp.concatenate([user_idx, item_idx + n_users]).reshape(-1, SUB)
    g = _gather_kernel(n_node, 2 * batch, len(e_list))(*e_list, all_idx)
    e_user = jnp.concatenate([g[t, :batch] for t in range(len(e_list))], axis=1)
    e_item = jnp.concatenate([g[t, batch:] for t in range(len(e_list))], axis=1)
    return (e_user, e_item)


# speedup vs baseline: 11.2318x; 1.4837x over previous
import functools

import jax
import jax.numpy as jnp
from jax import lax
from jax.experimental import pallas as pl
from jax.experimental.pallas import tpu as pltpu
from jax.experimental.pallas import tpu_sc as plsc

D = 64
DH = D // 2
NEG = 0.2
NC, NS, L = 2, 16, 16

SUB = 128
NU = 2
K = SUB * NU
NB = 3

_SC_PARAMS = pltpu.CompilerParams(needs_layout_passes=False,
                                  use_tc_tiling_on_sc=False)


def _chunks(total, step):
    out, off = [], 0
    while off < total:
        out.append((off, min(step, total - off)))
        off += out[-1][1]
    return out


def _spmm_kernel(n_node, nnz_pad):
    ept = nnz_pad // NS
    nchunk = ept // K
    nmacro = nchunk // NB
    assert nchunk % NB == 0
    stripe = -(-n_node // NS)
    rpt_last = n_node - (NS - 1) * stripe
    assert 0 < rpt_last <= stripe

    mesh = plsc.VectorSubcoreMesh(core_axis_name="c", subcore_axis_name="s",
                                  num_cores=NC, num_subcores=NS)

    @functools.partial(
        pl.kernel,
        out_type=jax.ShapeDtypeStruct((NC, n_node, DH), jnp.float32),
        mesh=mesh,
        compiler_params=_SC_PARAMS,
        scratch_types=[
            pltpu.VMEM((NB, NU, SUB, DH), jnp.float32),
            pltpu.VMEM((NB, 3, NU, SUB), jnp.int32),
            pltpu.VMEM_SHARED((n_node, DH), jnp.float32),
        ] + [pltpu.SemaphoreType.DMA] * (3 * NB),
    )
    def spmm(e_hbm, coo_hbm, out_hbm, rows_v, coo_v, acc, *sems):
        semc, semg, sems_ = sems[0:NB], sems[NB:2 * NB], sems[2 * NB:3 * NB]
        cid = lax.axis_index("c")
        sid = lax.axis_index("s")
        cbase = sid * nchunk

        def zrow(r, _):
            for q in range(DH // L):
                rows_v[0, 0, r, pl.ds(q * L, L)] = jnp.zeros((L,), jnp.float32)
            return 0
        lax.fori_loop(0, SUB, zrow, 0)
        zc = _chunks(stripe, SUB)
        for zoff, zn in zc:
            pltpu.async_copy(rows_v.at[0, 0, pl.ds(0, zn)],
                             acc.at[pl.ds(sid * stripe + zoff, zn)], semc[0])
        for zoff, zn in zc:
            pltpu.make_async_copy(rows_v.at[0, 0, pl.ds(0, zn)],
                                  acc.at[pl.ds(sid * stripe + zoff, zn)],
                                  semc[0]).wait()
        plsc.subcore_barrier()

        def process(b):
            for u in range(NU):
                def scale(i, _):
                    vv = plsc.bitcast(coo_v[b, 2, u, pl.ds(i * L, L)],
                                      jnp.float32)
                    for j in range(L):
                        v = jnp.broadcast_to(vv[j], (L,))
                        for q in range(DH // L):
                            rows_v[b, u, i * L + j, pl.ds(q * L, L)] = (
                                rows_v[b, u, i * L + j, pl.ds(q * L, L)] * v)
                    return 0
                lax.fori_loop(0, SUB // L, scale, 0)
            for u in range(NU):
                pltpu.async_copy(rows_v.at[b, u],
                                 acc.at[coo_v.at[b, 1, u]], sems_[b],
                                 add=True)

        def fire_gathers(b):
            for u in range(NU):
                pltpu.async_copy(e_hbm.at[cid].at[coo_v.at[b, 0, u]],
                                 rows_v.at[b, u], semg[b])

        def wait_gathers(b):
            for u in range(NU):
                pltpu.make_async_copy(e_hbm.at[0].at[coo_v.at[b, 0, u]],
                                      rows_v.at[b, u], semg[b]).wait()

        def wait_scatters(b):
            for u in range(NU):
                pltpu.make_async_copy(rows_v.at[b, u],
                                      acc.at[coo_v.at[b, 1, u]],
                                      sems_[b]).wait()

        pltpu.async_copy(coo_hbm.at[cbase], coo_v.at[0], semc[0])
        pltpu.async_copy(coo_hbm.at[cbase + 1], coo_v.at[1], semc[1])

        def macro(g2, _):
            for b in range(NB):
                g = g2 * NB + b
                pb = (b + NB - 1) % NB
                pltpu.make_async_copy(coo_hbm.at[cbase],
                                      coo_v.at[b], semc[b]).wait()
                @pl.when(g2 >= 1)
                def _():
                    wait_scatters(b)
                fire_gathers(b)
                if b == 0:
                    @pl.when(g2 >= 1)
                    def _():
                        wait_gathers(pb)
                        process(pb)
                else:
                    wait_gathers(pb)
                    process(pb)
                nxt = lax.min(cbase + g + 2, cbase + nchunk - 1)
                if b == 0:
                    pltpu.async_copy(coo_hbm.at[nxt], coo_v.at[pb], semc[pb])
                else:
                    @pl.when(g2 < nmacro - 1)
                    def _():
                        pltpu.async_copy(coo_hbm.at[nxt], coo_v.at[pb],
                                         semc[pb])
            return 0
        lax.fori_loop(0, nmacro, macro, 0)

        lb = (nchunk - 1) % NB
        wait_gathers(lb)
        process(lb)
        for b in range(NB):
            wait_scatters(b)
        plsc.subcore_barrier()

        base = sid * stripe

        def copy_out(total):
            cks = _chunks(total, SUB)
            for idx, (coff, cn) in enumerate(cks):
                sl = idx & 1
                if idx >= 2:
                    poff, pcn = cks[idx - 2]
                    pltpu.make_async_copy(
                        rows_v.at[0, sl, pl.ds(0, pcn)],
                        out_hbm.at[cid, pl.ds(base + poff, pcn)],
                        semg[sl]).wait()
                pltpu.sync_copy(acc.at[pl.ds(base + coff, cn)],
                                rows_v.at[0, sl, pl.ds(0, cn)])
                pltpu.async_copy(rows_v.at[0, sl, pl.ds(0, cn)],
                                 out_hbm.at[cid, pl.ds(base + coff, cn)],
                                 semg[sl])
            for idx in range(max(0, len(cks) - 2), len(cks)):
                coff, cn = cks[idx]
                pltpu.make_async_copy(
                    rows_v.at[0, idx & 1, pl.ds(0, cn)],
                    out_hbm.at[cid, pl.ds(base + coff, cn)],
                    semg[idx & 1]).wait()
        @pl.when(sid < NS - 1)
        def _():
            copy_out(stripe)
        @pl.when(sid == NS - 1)
        def _():
            copy_out(rpt_last)

    return spmm


def _dense_kernel(n_node):
    blk = 2000
    assert n_node % blk == 0

    def body(side_ref, e_ref, w1_ref, w2_ref, out_ref):
        s = jnp.concatenate([side_ref[0], side_ref[1]], axis=1)
        e = jnp.concatenate([e_ref[0], e_ref[1]], axis=1)
        y = (jnp.dot(s + e, w1_ref[...], preferred_element_type=jnp.float32)
             + jnp.dot(e * s, w2_ref[...], preferred_element_type=jnp.float32))
        y = jnp.where(y >= 0, y, NEG * y)
        out_ref[0] = y[:, :DH]
        out_ref[1] = y[:, DH:]

    return pl.pallas_call(
        body,
        grid=(n_node // blk,),
        in_specs=[
            pl.BlockSpec((NC, blk, DH), lambda i: (0, i, 0)),
            pl.BlockSpec((NC, blk, DH), lambda i: (0, i, 0)),
            pl.BlockSpec((D, D), lambda i: (0, 0)),
            pl.BlockSpec((D, D), lambda i: (0, 0)),
        ],
        out_specs=pl.BlockSpec((NC, blk, DH), lambda i: (0, i, 0)),
        out_shape=jax.ShapeDtypeStruct((NC, n_node, DH), jnp.float32),
    )


def _gather_kernel(n_node, n_idx, n_tab):
    nw = NC * NS
    per_w = n_idx // nw
    gsub = per_w // SUB
    assert per_w % SUB == 0

    mesh = plsc.VectorSubcoreMesh(core_axis_name="c", subcore_axis_name="s",
                                  num_cores=NC, num_subcores=NS)

    @functools.partial(
        pl.kernel,
        out_type=jax.ShapeDtypeStruct((n_tab, NC, n_idx, DH), jnp.float32),
        mesh=mesh,
        compiler_params=_SC_PARAMS,
        scratch_types=[
            pltpu.VMEM((gsub, SUB), jnp.int32),
            pltpu.VMEM((per_w, DH), jnp.float32),
            pltpu.SemaphoreType.DMA,
        ],
    )
    def gather(t0, t1, t2, t3, idx_hbm, out_hbm, idx_v, rows_v, sem):
        wid = lax.axis_index("s") * NC + lax.axis_index("c")
        base = wid * per_w
        pltpu.sync_copy(idx_hbm.at[pl.ds(wid * gsub, gsub)], idx_v)
        for t, tab in enumerate((t0, t1, t2, t3)):
            for h in range(NC):
                cps = [pltpu.async_copy(tab.at[h].at[idx_v.at[s]],
                                        rows_v.at[pl.ds(s * SUB, SUB)], sem)
                       for s in range(gsub)]
                for cp in cps:
                    cp.wait()
                pltpu.sync_copy(rows_v, out_hbm.at[t, h, pl.ds(base, per_w)])

    return gather


def kernel(user_idx, item_idx, emb, W1, W2, L_row, L_col, L_val):
    n_node, d = emb.shape
    n_layer = W1.shape[0]
    nnz = L_row.shape[0]
    assert d == D

    nnz_pad = -(-nnz // (NS * K * NB)) * (NS * K * NB)
    pad = nnz_pad - nnz
    if pad:
        spread = (jnp.arange(pad, dtype=jnp.int32) * 67) % n_node
        L_row = jnp.concatenate([L_row, spread])
        L_col = jnp.concatenate([L_col, spread])
        L_val = jnp.concatenate([L_val, jnp.zeros((pad,), jnp.float32)])
    val_bits = lax.bitcast_convert_type(L_val, jnp.int32)
    coo = jnp.stack([L_col.reshape(-1, NU, SUB), L_row.reshape(-1, NU, SUB),
                     val_bits.reshape(-1, NU, SUB)], axis=1)

    spmm = _spmm_kernel(n_node, nnz_pad)
    dense = _dense_kernel(n_node)

    e_prev = jnp.stack([emb[:, :DH], emb[:, DH:]], axis=0)
    e_list = [e_prev]
    for layer in range(n_layer):
        side = spmm(e_prev, coo)
        e_prev = dense(side, e_prev, W1[layer], W2[layer])
        e_list.append(e_prev)

    n_users = n_node // 2
    batch = user_idx.shape[0]
    all_idx = jnp.concatenate([user_idx, item_idx + n_users]).reshape(-1, SUB)
    g = _gather_kernel(n_node, 2 * batch, len(e_list))(*e_list, all_idx)
    e_user = jnp.concatenate(
        [g[t, h, :batch] for t in range(len(e_list)) for h in range(NC)],
        axis=1)
    e_item = jnp.concatenate(
        [g[t, h, batch:] for t in range(len(e_list)) for h in range(NC)],
        axis=1)
    return (e_user, e_item)


# speedup vs baseline: 11.3843x; 1.0136x over previous
import functools

import jax
import jax.numpy as jnp
from jax import lax
from jax.experimental import pallas as pl
from jax.experimental.pallas import tpu as pltpu
from jax.experimental.pallas import tpu_sc as plsc

D = 64
DH = D // 2
NEG = 0.2
NC, NS, L = 2, 16, 16

SUB = 128
NU = 2
K = SUB * NU
NB = 3

_SC_PARAMS = pltpu.CompilerParams(needs_layout_passes=False,
                                  use_tc_tiling_on_sc=False)


def _chunks(total, step):
    out, off = [], 0
    while off < total:
        out.append((off, min(step, total - off)))
        off += out[-1][1]
    return out


def _spmm_kernel(n_node, nnz_pad):
    ept = nnz_pad // NS
    nchunk = ept // K
    nmacro = nchunk // NB
    assert nchunk % NB == 0
    stripe = -(-n_node // NS)
    rpt_last = n_node - (NS - 1) * stripe
    assert 0 < rpt_last <= stripe

    mesh = plsc.VectorSubcoreMesh(core_axis_name="c", subcore_axis_name="s",
                                  num_cores=NC, num_subcores=NS)

    @functools.partial(
        pl.kernel,
        out_type=jax.ShapeDtypeStruct((NC, n_node, DH), jnp.float32),
        mesh=mesh,
        compiler_params=_SC_PARAMS,
        scratch_types=[
            pltpu.VMEM((NB, NU, SUB, DH), jnp.float32),
            pltpu.VMEM((NB, 3, NU, SUB), jnp.int32),
            pltpu.VMEM_SHARED((n_node, DH), jnp.float32),
        ] + [pltpu.SemaphoreType.DMA] * (3 * NB),
    )
    def spmm(e_hbm, coo_hbm, out_hbm, rows_v, coo_v, acc, *sems):
        semc, semg, sems_ = sems[0:NB], sems[NB:2 * NB], sems[2 * NB:3 * NB]
        cid = lax.axis_index("c")
        sid = lax.axis_index("s")
        cbase = sid * nchunk

        def zrow(r, _):
            for q in range(DH // L):
                rows_v[0, 0, r, pl.ds(q * L, L)] = jnp.zeros((L,), jnp.float32)
            return 0
        lax.fori_loop(0, SUB, zrow, 0)
        zc = _chunks(stripe, SUB)
        for zoff, zn in zc:
            pltpu.async_copy(rows_v.at[0, 0, pl.ds(0, zn)],
                             acc.at[pl.ds(sid * stripe + zoff, zn)], semc[0])
        for zoff, zn in zc:
            pltpu.make_async_copy(rows_v.at[0, 0, pl.ds(0, zn)],
                                  acc.at[pl.ds(sid * stripe + zoff, zn)],
                                  semc[0]).wait()
        plsc.subcore_barrier()

        def process(b):
            for u in range(NU):
                @plsc.parallel_loop(0, SUB // L, unroll=2)
                def scale(i):
                    vv = plsc.bitcast(coo_v[b, 2, u, pl.ds(i * L, L)],
                                      jnp.float32)
                    for j in range(L):
                        v = jnp.broadcast_to(vv[j], (L,))
                        for q in range(DH // L):
                            rows_v[b, u, i * L + j, pl.ds(q * L, L)] = (
                                rows_v[b, u, i * L + j, pl.ds(q * L, L)] * v)
            for u in range(NU):
                pltpu.async_copy(rows_v.at[b, u],
                                 acc.at[coo_v.at[b, 1, u]], sems_[b],
                                 add=True)

        def fire_gathers(b):
            for u in range(NU):
                pltpu.async_copy(e_hbm.at[cid].at[coo_v.at[b, 0, u]],
                                 rows_v.at[b, u], semg[b])

        def wait_gathers(b):
            for u in range(NU):
                pltpu.make_async_copy(e_hbm.at[0].at[coo_v.at[b, 0, u]],
                                      rows_v.at[b, u], semg[b]).wait()

        def wait_scatters(b):
            for u in range(NU):
                pltpu.make_async_copy(rows_v.at[b, u],
                                      acc.at[coo_v.at[b, 1, u]],
                                      sems_[b]).wait()

        pltpu.async_copy(coo_hbm.at[cbase], coo_v.at[0], semc[0])
        pltpu.async_copy(coo_hbm.at[cbase + 1], coo_v.at[1], semc[1])

        def macro(g2, _):
            for b in range(NB):
                g = g2 * NB + b
                pb = (b + NB - 1) % NB
                pltpu.make_async_copy(coo_hbm.at[cbase],
                                      coo_v.at[b], semc[b]).wait()
                @pl.when(g2 >= 1)
                def _():
                    wait_scatters(b)
                fire_gathers(b)
                if b == 0:
                    @pl.when(g2 >= 1)
                    def _():
                        wait_gathers(pb)
                        process(pb)
                else:
                    wait_gathers(pb)
                    process(pb)
                nxt = lax.min(cbase + g + 2, cbase + nchunk - 1)
                if b == 0:
                    pltpu.async_copy(coo_hbm.at[nxt], coo_v.at[pb], semc[pb])
                else:
                    @pl.when(g2 < nmacro - 1)
                    def _():
                        pltpu.async_copy(coo_hbm.at[nxt], coo_v.at[pb],
                                         semc[pb])
            return 0
        lax.fori_loop(0, nmacro, macro, 0)

        lb = (nchunk - 1) % NB
        wait_gathers(lb)
        process(lb)
        for b in range(NB):
            wait_scatters(b)
        plsc.subcore_barrier()

        base = sid * stripe

        def copy_out(total):
            cks = _chunks(total, SUB)
            for idx, (coff, cn) in enumerate(cks):
                sl = idx & 1
                if idx >= 2:
                    poff, pcn = cks[idx - 2]
                    pltpu.make_async_copy(
                        rows_v.at[0, sl, pl.ds(0, pcn)],
                        out_hbm.at[cid, pl.ds(base + poff, pcn)],
                        semg[sl]).wait()
                pltpu.sync_copy(acc.at[pl.ds(base + coff, cn)],
                                rows_v.at[0, sl, pl.ds(0, cn)])
                pltpu.async_copy(rows_v.at[0, sl, pl.ds(0, cn)],
                                 out_hbm.at[cid, pl.ds(base + coff, cn)],
                                 semg[sl])
            for idx in range(max(0, len(cks) - 2), len(cks)):
                coff, cn = cks[idx]
                pltpu.make_async_copy(
                    rows_v.at[0, idx & 1, pl.ds(0, cn)],
                    out_hbm.at[cid, pl.ds(base + coff, cn)],
                    semg[idx & 1]).wait()
        @pl.when(sid < NS - 1)
        def _():
            copy_out(stripe)
        @pl.when(sid == NS - 1)
        def _():
            copy_out(rpt_last)

    return spmm


def _dense_kernel(n_node):
    blk = 2000
    assert n_node % blk == 0

    def body(side_ref, e_ref, w1_ref, w2_ref, out_ref):
        s = jnp.concatenate([side_ref[0], side_ref[1]], axis=1)
        e = jnp.concatenate([e_ref[0], e_ref[1]], axis=1)
        y = (jnp.dot(s + e, w1_ref[...], preferred_element_type=jnp.float32)
             + jnp.dot(e * s, w2_ref[...], preferred_element_type=jnp.float32))
        y = jnp.where(y >= 0, y, NEG * y)
        out_ref[0] = y[:, :DH]
        out_ref[1] = y[:, DH:]

    return pl.pallas_call(
        body,
        grid=(n_node // blk,),
        in_specs=[
            pl.BlockSpec((NC, blk, DH), lambda i: (0, i, 0)),
            pl.BlockSpec((NC, blk, DH), lambda i: (0, i, 0)),
            pl.BlockSpec((D, D), lambda i: (0, 0)),
            pl.BlockSpec((D, D), lambda i: (0, 0)),
        ],
        out_specs=pl.BlockSpec((NC, blk, DH), lambda i: (0, i, 0)),
        out_shape=jax.ShapeDtypeStruct((NC, n_node, DH), jnp.float32),
    )


def _gather_kernel(n_node, n_idx, n_tab):
    nw = NC * NS
    per_w = n_idx // nw
    gsub = per_w // SUB
    assert per_w % SUB == 0

    mesh = plsc.VectorSubcoreMesh(core_axis_name="c", subcore_axis_name="s",
                                  num_cores=NC, num_subcores=NS)

    @functools.partial(
        pl.kernel,
        out_type=jax.ShapeDtypeStruct((n_tab, NC, n_idx, DH), jnp.float32),
        mesh=mesh,
        compiler_params=_SC_PARAMS,
        scratch_types=[
            pltpu.VMEM((gsub, SUB), jnp.int32),
            pltpu.VMEM((per_w, DH), jnp.float32),
            pltpu.SemaphoreType.DMA,
        ],
    )
    def gather(t0, t1, t2, t3, idx_hbm, out_hbm, idx_v, rows_v, sem):
        wid = lax.axis_index("s") * NC + lax.axis_index("c")
        base = wid * per_w
        pltpu.sync_copy(idx_hbm.at[pl.ds(wid * gsub, gsub)], idx_v)
        for t, tab in enumerate((t0, t1, t2, t3)):
            for h in range(NC):
                cps = [pltpu.async_copy(tab.at[h].at[idx_v.at[s]],
                                        rows_v.at[pl.ds(s * SUB, SUB)], sem)
                       for s in range(gsub)]
                for cp in cps:
                    cp.wait()
                pltpu.sync_copy(rows_v, out_hbm.at[t, h, pl.ds(base, per_w)])

    return gather


def kernel(user_idx, item_idx, emb, W1, W2, L_row, L_col, L_val):
    n_node, d = emb.shape
    n_layer = W1.shape[0]
    nnz = L_row.shape[0]
    assert d == D

    nnz_pad = -(-nnz // (NS * K * NB)) * (NS * K * NB)
    pad = nnz_pad - nnz
    if pad:
        spread = (jnp.arange(pad, dtype=jnp.int32) * 67) % n_node
        L_row = jnp.concatenate([L_row, spread])
        L_col = jnp.concatenate([L_col, spread])
        L_val = jnp.concatenate([L_val, jnp.zeros((pad,), jnp.float32)])
    val_bits = lax.bitcast_convert_type(L_val, jnp.int32)
    coo = jnp.stack([L_col.reshape(-1, NU, SUB), L_row.reshape(-1, NU, SUB),
                     val_bits.reshape(-1, NU, SUB)], axis=1)

    spmm = _spmm_kernel(n_node, nnz_pad)
    dense = _dense_kernel(n_node)

    e_prev = jnp.stack([emb[:, :DH], emb[:, DH:]], axis=0)
    e_list = [e_prev]
    for layer in range(n_layer):
        side = spmm(e_prev, coo)
        e_prev = dense(side, e_prev, W1[layer], W2[layer])
        e_list.append(e_prev)

    n_users = n_node // 2
    batch = user_idx.shape[0]
    all_idx = jnp.concatenate([user_idx, item_idx + n_users]).reshape(-1, SUB)
    g = _gather_kernel(n_node, 2 * batch, len(e_list))(*e_list, all_idx)
    e_user = jnp.concatenate(
        [g[t, h, :batch] for t in range(len(e_list)) for h in range(NC)],
        axis=1)
    e_item = jnp.concatenate(
        [g[t, h, batch:] for t in range(len(e_list)) for h in range(NC)],
        axis=1)
    return (e_user, e_item)
